# Initial kernel scaffold; baseline (speedup 1.0000x reference)
import jax
import jax.numpy as jnp
from jax.experimental import pallas as pl


def kernel(user_emb, entity_emb, item_emb_cf, relation_weight, gate1_w, gate2_w, mat_val, edge_index, edge_type, mat_row, mat_col):
    raise NotImplementedError("write your pallas kernel here")


# speedup vs baseline: 1.4084x; 1.4084x over previous
import functools

import jax
import jax.numpy as jnp
from jax.experimental import pallas as pl

EPS = 1e-10
MAX_NORM = 1.0 - 1e-5


def _rownorm(x):
    return jnp.sqrt(jnp.sum(x * x, axis=-1, keepdims=True))


def _project(x):
    norm = jnp.maximum(_rownorm(x), EPS)
    scale = jnp.where(norm > MAX_NORM, MAX_NORM / norm, 1.0)
    return x * scale


def _mobius_add(x, y):
    x2 = jnp.sum(x * x, axis=-1, keepdims=True)
    y2 = jnp.sum(y * y, axis=-1, keepdims=True)
    xy = jnp.sum(x * y, axis=-1, keepdims=True)
    num = (1.0 + 2.0 * xy + y2) * x + (1.0 - x2) * y
    den = jnp.maximum(1.0 + 2.0 * xy + x2 * y2, EPS)
    return num / den


def _edge_math_block(head_emb, tail_emb, rel_emb):
    un_h = jnp.maximum(_rownorm(head_emb), EPS)
    hh = _project(jnp.tanh(un_h) * head_emb / un_h)
    hh2 = jnp.sum(hh * hh, axis=-1, keepdims=True)
    lam = 2.0 / jnp.maximum(1.0 - hh2, EPS)
    half_lam = 0.5 * lam

    un_t = jnp.maximum(_rownorm(tail_emb), EPS)
    sec_t = jnp.tanh(half_lam * un_t) * tail_emb / un_t
    ht = _project(_mobius_add(hh, sec_t))

    un_r = jnp.maximum(_rownorm(rel_emb), EPS)
    sec_r = jnp.tanh(half_lam * un_r) * rel_emb / un_r
    hr = _project(_mobius_add(hh, sec_r))

    m = _project(_mobius_add(ht, hr))

    sub = _mobius_add(-hh, m)
    sn = jnp.maximum(jnp.maximum(_rownorm(sub), EPS), EPS)
    snc = jnp.clip(sn, -MAX_NORM, MAX_NORM)
    artanh = 0.5 * (jnp.log1p(snc) - jnp.log1p(-snc))
    return (2.0 / lam) * artanh * sub / sn


def _edge_kernel(head_ref, tail_ref, et_ref, rw_ref, out_ref):
    head_emb = head_ref[...]
    tail_emb = tail_ref[...]
    rel = et_ref[0, 0, :] - 1
    oh = (rel[:, None] == jax.lax.iota(jnp.int32, 16)[None, :]).astype(jnp.float32)
    rel_emb = jnp.dot(oh, rw_ref[...], preferred_element_type=jnp.float32)
    out_ref[...] = _edge_math_block(head_emb, tail_emb, rel_emb)


def _pick_block(e, target=2000):
    best = 8
    for b in range(8, min(e, 4096) + 1, 8):
        if e % b == 0 and abs(b - target) < abs(best - target):
            best = b
    return best


@functools.partial(jax.jit, static_argnames=("interpret",))
def _edge_transform(head_emb, tail_emb, edge_type, rw_pad, interpret=False):
    e, c = head_emb.shape
    be = _pick_block(e)
    nb = e // be
    et3 = edge_type.reshape(nb, 1, be)
    return pl.pallas_call(
        _edge_kernel,
        grid=(nb,),
        in_specs=[
            pl.BlockSpec((be, c), lambda i: (i, 0)),
            pl.BlockSpec((be, c), lambda i: (i, 0)),
            pl.BlockSpec((1, 1, be), lambda i: (i, 0, 0)),
            pl.BlockSpec((16, c), lambda i: (0, 0)),
        ],
        out_specs=pl.BlockSpec((be, c), lambda i: (i, 0)),
        out_shape=jax.ShapeDtypeStruct((e, c), jnp.float32),
        interpret=interpret,
    )(head_emb, tail_emb, et3, rw_pad)


def _l2norm(x):
    n = jnp.maximum(jnp.sqrt(jnp.sum(x * x, axis=-1, keepdims=True)), 1e-12)
    return x / n


def kernel(user_emb, entity_emb, item_emb_cf, relation_weight, gate1_w, gate2_w,
           mat_val, edge_index, edge_type, mat_row, mat_col, interpret=False):
    n_entities = entity_emb.shape[0]
    n_users = user_emb.shape[0]
    n_items = item_emb_cf.shape[0]
    c = entity_emb.shape[1]
    head = edge_index[0]
    tail = edge_index[1]

    rw_pad = jnp.zeros((16, c), jnp.float32).at[: relation_weight.shape[0]].set(relation_weight)

    e_res, u_res, i_res = entity_emb, user_emb, item_emb_cf
    cur_e, cur_u, cur_i = entity_emb, user_emb, item_emb_cf
    ones_e = jnp.ones((head.shape[0],), dtype=entity_emb.dtype)
    cnt = jax.ops.segment_sum(ones_e, head, num_segments=n_entities)
    inv_cnt = 1.0 / jnp.maximum(cnt, 1.0)
    n_hops = gate1_w.shape[0]
    for hop in range(n_hops):
        head_emb = cur_e[head]
        tail_emb = cur_e[tail]
        res = _edge_transform(head_emb, tail_emb, edge_type, rw_pad, interpret=interpret)
        sums = jax.ops.segment_sum(res, head, num_segments=n_entities)
        entity_agg = sums * inv_cnt[:, None]
        item_agg_cf = jax.ops.segment_sum(cur_u[mat_row], mat_col, num_segments=n_items)
        item_emb_kg = cur_e[:n_items]
        gi = jax.nn.sigmoid(cur_i @ gate1_w[hop].T + item_emb_kg @ gate2_w[hop].T)
        item_fusion = gi * cur_i + (1.0 - gi) * item_emb_kg
        user_agg = jax.ops.segment_sum(mat_val[:, None] * item_fusion[mat_col], mat_row, num_segments=n_users)
        cur_e = _l2norm(entity_agg)
        cur_u = _l2norm(user_agg)
        cur_i = _l2norm(item_agg_cf)
        e_res = e_res + cur_e
        u_res = u_res + cur_u
        i_res = i_res + cur_i
    return (e_res, u_res, i_res)


# speedup vs baseline: 2.6813x; 1.9037x over previous
import functools

import jax
import jax.numpy as jnp
from jax import lax
from jax.experimental import pallas as pl
from jax.experimental.pallas import tpu as pltpu
from jax.experimental.pallas import tpu_sc as plsc

EPS = 1e-10
MAX_NORM = 1.0 - 1e-5

NC = 2
NS = 16
LANES = 16

_MESH = dict(core_axis_name="c", subcore_axis_name="s", num_cores=NC,
             num_subcores=NS)


def _rownorm(x):
    return jnp.sqrt(jnp.sum(x * x, axis=-1, keepdims=True))


def _project(x):
    norm = jnp.maximum(_rownorm(x), EPS)
    scale = jnp.where(norm > MAX_NORM, MAX_NORM / norm, 1.0)
    return x * scale


def _mobius_add(x, y):
    x2 = jnp.sum(x * x, axis=-1, keepdims=True)
    y2 = jnp.sum(y * y, axis=-1, keepdims=True)
    xy = jnp.sum(x * y, axis=-1, keepdims=True)
    num = (1.0 + 2.0 * xy + y2) * x + (1.0 - x2) * y
    den = jnp.maximum(1.0 + 2.0 * xy + x2 * y2, EPS)
    return num / den


def _edge_math_block(head_emb, tail_emb, rel_emb):
    un_h = jnp.maximum(_rownorm(head_emb), EPS)
    hh = _project(jnp.tanh(un_h) * head_emb / un_h)
    hh2 = jnp.sum(hh * hh, axis=-1, keepdims=True)
    lam = 2.0 / jnp.maximum(1.0 - hh2, EPS)
    half_lam = 0.5 * lam

    un_t = jnp.maximum(_rownorm(tail_emb), EPS)
    ht = _project(_mobius_add(hh, jnp.tanh(half_lam * un_t) * tail_emb / un_t))
    un_r = jnp.maximum(_rownorm(rel_emb), EPS)
    hr = _project(_mobius_add(hh, jnp.tanh(half_lam * un_r) * rel_emb / un_r))

    m = _project(_mobius_add(ht, hr))
    sub = _mobius_add(-hh, m)
    sn = jnp.maximum(_rownorm(sub), EPS)
    snc = jnp.clip(sn, -MAX_NORM, MAX_NORM)
    artanh = 0.5 * (jnp.log1p(snc) - jnp.log1p(-snc))
    return (2.0 / lam) * artanh * sub / sn


def _edge_kernel(head_ref, tail_ref, et_ref, rw_ref, out_ref):
    rel = et_ref[0, 0, :] - 1
    oh = (rel[:, None] == lax.iota(jnp.int32, 16)[None, :]).astype(jnp.float32)
    rel_emb = jnp.dot(oh, rw_ref[...], preferred_element_type=jnp.float32)
    out_ref[...] = _edge_math_block(head_ref[...], tail_ref[...], rel_emb)


def _pick_block(e, target=2000):
    best = 8
    for b in range(8, min(e, 4096) + 1, 8):
        if e % b == 0 and abs(b - target) < abs(best - target):
            best = b
    return best


def _edge_transform(head_emb, tail_emb, edge_type, rw_pad):
    e, c = head_emb.shape
    be = _pick_block(e)
    nb = e // be
    et3 = edge_type.reshape(nb, 1, be)
    return pl.pallas_call(
        _edge_kernel,
        grid=(nb,),
        in_specs=[
            pl.BlockSpec((be, c), lambda i: (i, 0)),
            pl.BlockSpec((be, c), lambda i: (i, 0)),
            pl.BlockSpec((1, 1, be), lambda i: (i, 0, 0)),
            pl.BlockSpec((16, c), lambda i: (0, 0)),
        ],
        out_specs=pl.BlockSpec((be, c), lambda i: (i, 0)),
        out_shape=jax.ShapeDtypeStruct((e, c), jnp.float32),
    )(head_emb, tail_emb, et3, rw_pad)


def _wid():
    return lax.axis_index("s") * NC + lax.axis_index("c")


def _sc_gather2(table, idx_a, idx_b):
    e = idx_a.shape[0]
    c = table.shape[1]
    k = 1000
    per_w = e // (NC * NS)
    n_ch = per_w // k
    assert per_w % k == 0

    @functools.partial(
        pl.kernel,
        mesh=plsc.VectorSubcoreMesh(**_MESH),
        compiler_params=pltpu.CompilerParams(use_tc_tiling_on_sc=False),
        out_type=[jax.ShapeDtypeStruct((e, c), jnp.float32),
                  jax.ShapeDtypeStruct((e, c), jnp.float32)],
        scratch_types=[
            pltpu.VMEM((k,), jnp.int32),
            pltpu.VMEM((k, c), jnp.float32),
            pltpu.SemaphoreType.DMA,
        ],
    )
    def body(table_h, ia_h, ib_h, oa_h, ob_h, idx_v, rows_v, sem):
        base0 = _wid() * per_w

        def step(j, _):
            base = base0 + j * k
            pltpu.sync_copy(ia_h.at[pl.ds(base, k)], idx_v)
            pltpu.async_copy(table_h.at[idx_v], rows_v, sem).wait()
            pltpu.sync_copy(rows_v, oa_h.at[pl.ds(base, k)])
            pltpu.sync_copy(ib_h.at[pl.ds(base, k)], idx_v)
            pltpu.async_copy(table_h.at[idx_v], rows_v, sem).wait()
            pltpu.sync_copy(rows_v, ob_h.at[pl.ds(base, k)])
            return 0

        lax.fori_loop(0, n_ch, step, 0)

    return body(table, idx_a, idx_b)


def _translate_loop(idx_v, idx2_v, k, half, core, s, dspread=984):
    def tr(i, _):
        v = idx_v[pl.ds(i * LANES, LANES)]
        li = v - core * half
        ok = (li >= 0) & (li < half)
        dummy = half + ((i * LANES + s * 64) % dspread) + lax.iota(jnp.int32, LANES)
        idx2_v[pl.ds(i * LANES, LANES)] = jnp.where(ok, li, dummy)
        return 0

    lax.fori_loop(0, k // LANES, tr, 0)


def _sc_segsum_entity(res, head, zeros64):
    e, c = res.shape
    n_ent = 50000
    half = n_ent // NC
    hpad = half + 1000
    k = 400
    per_s = e // NS
    n_ch = per_s // k
    assert per_s % k == 0
    zk = 1000

    @functools.partial(
        pl.kernel,
        mesh=plsc.VectorSubcoreMesh(**_MESH),
        compiler_params=pltpu.CompilerParams(use_tc_tiling_on_sc=False),
        out_type=jax.ShapeDtypeStruct((n_ent, c), jnp.float32),
        scratch_types=[
            pltpu.VMEM((k,), jnp.int32),
            pltpu.VMEM((k,), jnp.int32),
            pltpu.VMEM((k, c), jnp.float32),
            pltpu.VMEM_SHARED((hpad, c), jnp.float32),
        ],
    )
    def body(res_h, head_h, zeros_h, out_h, idx_v, idx2_v, rows_v, acc):
        core = lax.axis_index("c")
        s = lax.axis_index("s")

        def zstep(i, _):
            j = s + i * NS
            @pl.when(j < hpad // zk)
            def _():
                pltpu.sync_copy(zeros_h, acc.at[pl.ds(j * zk, zk)])
            return 0
        lax.fori_loop(0, (hpad // zk + NS - 1) // NS, zstep, 0)
        plsc.subcore_barrier()

        def step(j, _):
            base = s * per_s + j * k
            pltpu.sync_copy(head_h.at[pl.ds(base, k)], idx_v)
            _translate_loop(idx_v, idx2_v, k, half, core, s)
            pltpu.sync_copy(res_h.at[pl.ds(base, k)], rows_v)
            pltpu.sync_copy(rows_v, acc.at[idx2_v], add=True)
            return 0
        lax.fori_loop(0, n_ch, step, 0)
        plsc.subcore_barrier()

        def ostep(i, _):
            j = s + i * NS
            @pl.when(j < half // zk)
            def _():
                pltpu.sync_copy(acc.at[pl.ds(j * zk, zk)],
                                out_h.at[pl.ds(core * half + j * zk, zk)])
            return 0
        lax.fori_loop(0, (half // zk + NS - 1) // NS, ostep, 0)

    return body(res, head, zeros64)


def _sc_counts(head, zeros16):
    e = head.shape[0]
    n_ent = 50000
    half = n_ent // NC
    hpad = half + 1000
    k = 400
    per_s = e // NS
    n_ch = per_s // k
    zk = 1000

    @functools.partial(
        pl.kernel,
        mesh=plsc.VectorSubcoreMesh(**_MESH),
        compiler_params=pltpu.CompilerParams(use_tc_tiling_on_sc=False),
        out_type=jax.ShapeDtypeStruct((n_ent, 16), jnp.float32),
        scratch_types=[
            pltpu.VMEM((k,), jnp.int32),
            pltpu.VMEM((k,), jnp.int32),
            pltpu.VMEM((k, 16), jnp.float32),
            pltpu.VMEM_SHARED((hpad, 16), jnp.float32),
        ],
    )
    def body(head_h, zeros_h, out_h, idx_v, idx2_v, ones_v, acc):
        core = lax.axis_index("c")
        s = lax.axis_index("s")

        def fill(r, _):
            ones_v[r, pl.ds(0, 16)] = jnp.full((16,), 1.0, jnp.float32)
            return 0
        lax.fori_loop(0, k, fill, 0)

        def zstep(i, _):
            j = s + i * NS
            @pl.when(j < hpad // zk)
            def _():
                pltpu.sync_copy(zeros_h, acc.at[pl.ds(j * zk, zk)])
            return 0
        lax.fori_loop(0, (hpad // zk + NS - 1) // NS, zstep, 0)
        plsc.subcore_barrier()

        def step(j, _):
            base = s * per_s + j * k
            pltpu.sync_copy(head_h.at[pl.ds(base, k)], idx_v)
            _translate_loop(idx_v, idx2_v, k, half, core, s)
            pltpu.sync_copy(ones_v, acc.at[idx2_v], add=True)
            return 0
        lax.fori_loop(0, n_ch, step, 0)
        plsc.subcore_barrier()

        def ostep(i, _):
            j = s + i * NS
            @pl.when(j < half // zk)
            def _():
                pltpu.sync_copy(acc.at[pl.ds(j * zk, zk)],
                                out_h.at[pl.ds(core * half + j * zk, zk)])
            return 0
        lax.fori_loop(0, (half // zk + NS - 1) // NS, ostep, 0)

    return body(head, zeros16)


def _sc_item_agg(user_tab, row_g, col_s, zeros64):
    nnzp = row_g.shape[0]
    c = user_tab.shape[1]
    n_items = 20000
    ipad = n_items + 1000
    k = 520
    per_w = nnzp // (NC * NS)
    n_ch = per_w // k
    assert per_w % k == 0
    zk = 1000

    @functools.partial(
        pl.kernel,
        mesh=plsc.VectorSubcoreMesh(**_MESH),
        compiler_params=pltpu.CompilerParams(use_tc_tiling_on_sc=False),
        out_type=jax.ShapeDtypeStruct((NC, n_items, c), jnp.float32),
        scratch_types=[
            pltpu.VMEM((k,), jnp.int32),
            pltpu.VMEM((k, c), jnp.float32),
            pltpu.VMEM_SHARED((ipad, c), jnp.float32),
            pltpu.SemaphoreType.DMA,
        ],
    )
    def body(tab_h, rg_h, cs_h, zeros_h, out_h, idx_v, rows_v, acc, sem):
        core = lax.axis_index("c")
        s = lax.axis_index("s")

        def zstep(i, _):
            j = s + i * NS
            @pl.when(j < ipad // zk)
            def _():
                pltpu.sync_copy(zeros_h, acc.at[pl.ds(j * zk, zk)])
            return 0
        lax.fori_loop(0, (ipad // zk + NS - 1) // NS, zstep, 0)
        plsc.subcore_barrier()

        def step(j, _):
            base = (core * NS + s) * per_w + j * k
            pltpu.sync_copy(rg_h.at[pl.ds(base, k)], idx_v)
            pltpu.async_copy(tab_h.at[idx_v], rows_v, sem).wait()
            pltpu.sync_copy(cs_h.at[pl.ds(base, k)], idx_v)
            pltpu.sync_copy(rows_v, acc.at[idx_v], add=True)
            return 0
        lax.fori_loop(0, n_ch, step, 0)
        plsc.subcore_barrier()

        def ostep(i, _):
            j = s + i * NS
            @pl.when(j < n_items // zk)
            def _():
                pltpu.sync_copy(acc.at[pl.ds(j * zk, zk)],
                                out_h.at[core, pl.ds(j * zk, zk)])
            return 0
        lax.fori_loop(0, (n_items // zk + NS - 1) // NS, ostep, 0)

    return body(user_tab, row_g, col_s, zeros64)


def _sc_user_agg(fusion_tab, col_g, row_s, val_p, zeros64):
    nnzp = col_g.shape[0]
    c = fusion_tab.shape[1]
    n_users = 50000
    half = n_users // NC
    hpad = half + 600
    k = 400
    per_s = nnzp // NS
    n_ch = per_s // k
    assert per_s % k == 0
    zk = 1000

    @functools.partial(
        pl.kernel,
        mesh=plsc.VectorSubcoreMesh(**_MESH),
        compiler_params=pltpu.CompilerParams(use_tc_tiling_on_sc=False, needs_layout_passes=False),
        out_type=jax.ShapeDtypeStruct((n_users, c), jnp.float32),
        scratch_types=[
            pltpu.VMEM((k,), jnp.int32),
            pltpu.VMEM((k,), jnp.int32),
            pltpu.VMEM((k,), jnp.float32),
            pltpu.VMEM((k, c), jnp.float32),
            pltpu.VMEM_SHARED((hpad, c), jnp.float32),
            pltpu.SemaphoreType.DMA,
        ],
    )
    def body(tab_h, cg_h, rs_h, val_h, zeros_h, out_h,
             idx_v, idx2_v, val_v, rows_v, acc, sem):
        core = lax.axis_index("c")
        s = lax.axis_index("s")

        def zstep(i, _):
            j = s + i * NS
            @pl.when(j < hpad // zk)
            def _():
                pltpu.sync_copy(zeros_h, acc.at[pl.ds(j * zk, zk)])
            return 0
        lax.fori_loop(0, (hpad // zk + NS - 1) // NS, zstep, 0)
        plsc.subcore_barrier()

        def step(j, _):
            base = s * per_s + j * k
            pltpu.sync_copy(cg_h.at[pl.ds(base, k)], idx_v)
            pltpu.async_copy(tab_h.at[idx_v], rows_v, sem).wait()
            pltpu.sync_copy(val_h.at[pl.ds(base, k)], val_v)

            def scale(r, _):
                sv = plsc.load_gather(val_v, [jnp.zeros((16,), jnp.int32) + r])
                for q in range(4):
                    rows_v[r, pl.ds(q * 16, 16)] = rows_v[r, pl.ds(q * 16, 16)] * sv
                return 0
            lax.fori_loop(0, k, scale, 0)

            pltpu.sync_copy(rs_h.at[pl.ds(base, k)], idx_v)
            _translate_loop(idx_v, idx2_v, k, half, core, s, dspread=584)
            pltpu.sync_copy(rows_v, acc.at[idx2_v], add=True)
            return 0
        lax.fori_loop(0, n_ch, step, 0)
        plsc.subcore_barrier()

        def ostep(i, _):
            j = s + i * NS
            @pl.when(j < half // zk)
            def _():
                pltpu.sync_copy(acc.at[pl.ds(j * zk, zk)],
                                out_h.at[pl.ds(core * half + j * zk, zk)])
            return 0
        lax.fori_loop(0, (half // zk + NS - 1) // NS, ostep, 0)

    return body(fusion_tab, col_g, row_s, val_p, zeros64)


def _l2norm(x):
    n = jnp.maximum(jnp.sqrt(jnp.sum(x * x, axis=-1, keepdims=True)), 1e-12)
    return x / n


def kernel(user_emb, entity_emb, item_emb_cf, relation_weight, gate1_w, gate2_w,
           mat_val, edge_index, edge_type, mat_row, mat_col):
    n_entities = entity_emb.shape[0]
    n_users = user_emb.shape[0]
    n_items = item_emb_cf.shape[0]
    c = entity_emb.shape[1]
    nnz = mat_row.shape[0]
    head = edge_index[0]
    tail = edge_index[1]

    rw_pad = jnp.zeros((16, c), jnp.float32).at[: relation_weight.shape[0]].set(relation_weight)
    zeros64 = jnp.zeros((1000, c), jnp.float32)
    zeros16 = jnp.zeros((1000, 16), jnp.float32)

    nnzp = 416000
    npad = nnzp - nnz
    ar = jnp.arange(npad, dtype=jnp.int32)
    row_g = jnp.concatenate([mat_row, ar % n_users])
    col_s = jnp.concatenate([mat_col, n_items + ar % 1000])
    col_g = jnp.concatenate([mat_col, ar % n_items])
    row_s = jnp.concatenate([mat_row, jnp.full((npad,), n_users, jnp.int32)])
    val_p = jnp.concatenate([mat_val, jnp.zeros((npad,), jnp.float32)])

    cnt16 = _sc_counts(head, zeros16)
    inv_cnt = 1.0 / jnp.maximum(cnt16[:, 0], 1.0)

    e_res, u_res, i_res = entity_emb, user_emb, item_emb_cf
    cur_e, cur_u, cur_i = entity_emb, user_emb, item_emb_cf
    n_hops = gate1_w.shape[0]
    for hop in range(n_hops):
        head_emb, tail_emb = _sc_gather2(cur_e, head, tail)
        res = _edge_transform(head_emb, tail_emb, edge_type, rw_pad)
        sums = _sc_segsum_entity(res, head, zeros64)
        entity_agg = sums * inv_cnt[:, None]
        item_parts = _sc_item_agg(cur_u, row_g, col_s, zeros64)
        item_agg_cf = item_parts[0] + item_parts[1]
        item_emb_kg = cur_e[:n_items]
        gi = jax.nn.sigmoid(cur_i @ gate1_w[hop].T + item_emb_kg @ gate2_w[hop].T)
        item_fusion = gi * cur_i + (1.0 - gi) * item_emb_kg
        user_agg = _sc_user_agg(item_fusion, col_g, row_s, val_p, zeros64)
        cur_e = _l2norm(entity_agg)
        cur_u = _l2norm(user_agg)
        cur_i = _l2norm(item_agg_cf)
        e_res = e_res + cur_e
        u_res = u_res + cur_u
        i_res = i_res + cur_i
    return (e_res, u_res, i_res)
